# baseline (speedup 1.0000x reference)
import functools

import jax
import jax.numpy as jnp
from jax import lax
from jax.experimental import pallas as pl
from jax.experimental.pallas import tpu as pltpu
from jax.experimental.pallas import tpu_sc as plsc

N = 10000
E = 160000
D = 256
H = 4
C = 64
BOT = 64
HC = H * C

NPAD = 10112
NROW = NPAD // 16
ETOT = E + N
EP = 172032
EP2 = 163840
K = 64
W1 = EP // 32
NCH1 = W1 // K
W4 = EP2 // 32
NCH4 = W4 // K

_f32 = jnp.float32
_i32 = jnp.int32

_MESH = dict(core_axis_name="c", subcore_axis_name="s")


def _wid():
    return lax.axis_index("s") * 2 + lax.axis_index("c")


NB = 8
RB = NPAD // NB


def _tc0_body(x_ref, wl_ref, wr_ref, xla_ref, xlb_ref, xra_ref, xrb_ref):
    x = x_ref[...]
    xl = jnp.dot(x, wl_ref[...], preferred_element_type=_f32,
                 precision=lax.Precision.DEFAULT)
    xr = jnp.dot(x, wr_ref[...], preferred_element_type=_f32,
                 precision=lax.Precision.DEFAULT)
    xla_ref[...] = xl[:, :128]
    xlb_ref[...] = xl[:, 128:]
    xra_ref[...] = xr[:, :128]
    xrb_ref[...] = xr[:, 128:]


def _tc1a_body(aa_ref, ab_ref, x_ref, bias_ref,
               hmid_ref, disb_ref, ssum_ref, ssq_ref):
    i = pl.program_id(0)

    @pl.when(i == 0)
    def _():
        ssum_ref[...] = jnp.zeros_like(ssum_ref)
        ssq_ref[...] = jnp.zeros_like(ssq_ref)

    a = aa_ref[0] + aa_ref[1]
    b = ab_ref[0] + ab_ref[1]
    d0 = a[:, 128:129]
    d1 = a[:, 129:130]
    d2 = b[:, 128:129]
    d3 = b[:, 129:130]
    deg = a[:, 130:131]
    gat = jnp.concatenate([
        jnp.where(d0 > 0, a[:, :64] / d0, 0.0),
        jnp.where(d1 > 0, a[:, 64:128] / d1, 0.0),
        jnp.where(d2 > 0, b[:, :64] / d2, 0.0),
        jnp.where(d3 > 0, b[:, 64:128] / d3, 0.0),
    ], axis=1) + bias_ref[...]
    hmid = jnp.maximum(gat, 0.0) + x_ref[...]
    hmid_ref[...] = hmid
    dis = jnp.where(deg > 0, lax.rsqrt(deg), 0.0)
    disb_ref[...] = jnp.broadcast_to(dis, (RB, BOT))
    rows = i * RB + lax.broadcasted_iota(_i32, (RB, 1), 0)
    hm = jnp.where(rows < N, hmid, 0.0)
    ssum_ref[...] += jnp.sum(hm, axis=0, keepdims=True)
    ssq_ref[...] += jnp.sum(hm * hm, axis=0, keepdims=True)


def _tc1b_body(hmid_ref, ssum_ref, ssq_ref, g1_ref, b1_ref, wg_ref, disb_ref,
               h_ref, gs_ref):
    mean = ssum_ref[...] / N
    var = ssq_ref[...] / N - mean * mean
    h = (hmid_ref[...] - mean) * lax.rsqrt(var + 1e-5) * g1_ref[...] \
        + b1_ref[...]
    h_ref[...] = h
    g1 = jnp.dot(h, wg_ref[...], preferred_element_type=_f32,
                 precision=lax.Precision.DEFAULT)
    gs_ref[...] = g1 * disb_ref[...]


def _tc2_body(a2_ref, disb_ref, bg_ref, g2_ref, b2_ref, z_ref):
    og = (a2_ref[0] + a2_ref[1]) * disb_ref[...] + bg_ref[...]
    r = jnp.maximum(og, 0.0)
    rows = lax.broadcasted_iota(_i32, (NPAD, 1), 0)
    mask = rows < N
    rm = jnp.where(mask, r, 0.0)
    mean = jnp.sum(rm, axis=0, keepdims=True) / N
    dev = jnp.where(mask, r - mean, 0.0)
    var = jnp.sum(dev * dev, axis=0, keepdims=True) / N
    z_ref[...] = (r - mean) * lax.rsqrt(var + 1e-5) * g2_ref[...] \
        + b2_ref[...]


def _sc1_body(s_hbm, d_hbm, xla_hbm, xlb_hbm, xra_hbm, xrb_hbm, att_hbm,
              logit_hbm, wmax_hbm,
              sv0, dv0, sv1, dv1,
              bla0, blb0, bra0, brb0, bla1, blb1, bra1, brb1,
              attv, lb, mbuf,
              s0a, s0b, s0c, s0d, s1a, s1b, s1c, s1d):
    wid = _wid()
    base = wid * W1
    pltpu.sync_copy(att_hbm, attv)
    iot = lax.iota(_i32, 16)
    zvec = jnp.zeros((16,), _f32)
    SLOT = (
        (sv0, dv0, (bla0, blb0, bra0, brb0), (s0a, s0b, s0c, s0d)),
        (sv1, dv1, (bla1, blb1, bra1, brb1), (s1a, s1b, s1c, s1d)),
    )

    def fire(g, slot):
        sv, dv, bufs, sems = SLOT[slot]
        off = base + g * K
        pltpu.sync_copy(s_hbm.at[pl.ds(off, K)], sv)
        pltpu.sync_copy(d_hbm.at[pl.ds(off, K)], dv)
        pltpu.async_copy(xla_hbm.at[sv], bufs[0], sems[0])
        pltpu.async_copy(xlb_hbm.at[sv], bufs[1], sems[1])
        pltpu.async_copy(xra_hbm.at[dv], bufs[2], sems[2])
        pltpu.async_copy(xrb_hbm.at[dv], bufs[3], sems[3])

    def drain(slot):
        sv, dv, bufs, sems = SLOT[slot]
        pltpu.make_async_copy(xla_hbm.at[sv], bufs[0], sems[0]).wait()
        pltpu.make_async_copy(xlb_hbm.at[sv], bufs[1], sems[1]).wait()
        pltpu.make_async_copy(xra_hbm.at[dv], bufs[2], sems[2]).wait()
        pltpu.make_async_copy(xrb_hbm.at[dv], bufs[3], sems[3]).wait()

    fire(0, 0)
    fire(1, 1)

    def chunk2(g2, mvec):
        for slot in range(2):
            g = g2 * 2 + slot
            off = base + g * K
            drain(slot)
            bla, blb, bra, brb = SLOT[slot][2]
            for h in range(H):
                bl, br = (bla, bra) if h < 2 else (blb, brb)
                cbase = (h % 2) * 64

                def feat(c2, accs):
                    asp = plsc.load_gather(
                        attv, [jnp.full((16,), h * 64 + c2, _i32)])
                    col = jnp.full((16,), cbase + c2, _i32)
                    out = []
                    for grp in range(K // 16):
                        ridx = grp * 16 + iot
                        u = (plsc.load_gather(bl, [ridx, col])
                             + plsc.load_gather(br, [ridx, col]))
                        u = jnp.maximum(u, 0.2 * u)
                        out.append(accs[grp] + asp * u)
                    return tuple(out)

                accs = lax.fori_loop(0, C, feat, (zvec,) * (K // 16),
                                     unroll=16)
                for grp in range(K // 16):
                    lb[h, pl.ds(grp * 16, 16)] = accs[grp]
                    mvec = jnp.maximum(mvec, accs[grp])
            pltpu.sync_copy(lb, logit_hbm.at[:, pl.ds(off, K)])

            @pl.when(g + 2 < NCH1)
            def _():
                fire(g + 2, slot)
        return mvec

    mvec = lax.fori_loop(0, NCH1 // 2, chunk2,
                         jnp.full((16,), -3.0e38, _f32))
    mbuf[...] = mvec
    pltpu.sync_copy(mbuf, wmax_hbm.at[pl.ds(wid * 16, 16)])


def _sc2_body(hoff, s_hbm, d_hbm, xh_hbm, logit_hbm, wmax_hbm, z144_hbm,
              acc_hbm,
              acc_sh, sv0, dv0, sv1, dv1, bx0, bx1,
              lb0, lb1, wb0, wb1, msg, mxv, sem0, sem1):
    c = lax.axis_index("c")
    t = lax.axis_index("s")
    wid = _wid()
    pltpu.sync_copy(wmax_hbm, mxv)
    mv = mxv[pl.ds(0, 16)]
    for v in range(1, 32):
        mv = jnp.maximum(mv, mxv[pl.ds(v * 16, 16)])
    gmax = jnp.max(mv)

    r0 = t * NROW
    pltpu.sync_copy(z144_hbm.at[pl.ds(r0, NROW)], acc_sh.at[pl.ds(r0, NROW)])
    plsc.subcore_barrier()

    base = wid * W1
    iot = lax.iota(_i32, 16)
    SLOT = ((sv0, dv0, bx0, sem0), (sv1, dv1, bx1, sem1))

    def fire(g, slot):
        sv, dv, bx, sem = SLOT[slot]
        off = base + g * K
        pltpu.sync_copy(s_hbm.at[pl.ds(off, K)], sv)
        pltpu.sync_copy(d_hbm.at[pl.ds(off, K)], dv)
        pltpu.async_copy(xh_hbm.at[sv], bx, sem)

    fire(0, 0)
    fire(1, 1)

    def chunk2(g2, _):
        for slot in range(2):
            g = g2 * 2 + slot
            off = base + g * K
            sv, dv, bx, sem = SLOT[slot]
            pltpu.sync_copy(logit_hbm.at[hoff, pl.ds(off, K)], lb0)
            pltpu.sync_copy(logit_hbm.at[hoff + 1, pl.ds(off, K)], lb1)
            pltpu.make_async_copy(xh_hbm.at[sv], bx, sem).wait()
            for v in range(K // 16):
                sl = pl.ds(v * 16, 16)
                wb0[sl] = jnp.exp(lb0[sl] - gmax)
                wb1[sl] = jnp.exp(lb1[sl] - gmax)

            def edge(j, _):
                jf = jnp.full((16,), j, _i32)
                w0 = plsc.load_gather(wb0, [jf])
                w1 = plsc.load_gather(wb1, [jf])
                for v in range(4):
                    msg[j, pl.ds(v * 16, 16)] = bx[j, pl.ds(v * 16, 16)] * w0
                for v in range(4, 8):
                    msg[j, pl.ds(v * 16, 16)] = bx[j, pl.ds(v * 16, 16)] * w1
                tail = jnp.where(iot == 0, w0,
                                 jnp.where(iot == 1, w1,
                                           jnp.where(iot == 2, 1.0, 0.0)))
                msg[j, pl.ds(128, 16)] = tail
                return 0

            lax.fori_loop(0, K, edge, 0, unroll=4)
            pltpu.sync_copy(msg, acc_sh.at[dv], add=True)

            @pl.when(g + 2 < NCH1)
            def _():
                fire(g + 2, slot)
        return 0

    lax.fori_loop(0, NCH1 // 2, chunk2, 0)
    plsc.subcore_barrier()
    pltpu.sync_copy(acc_sh.at[pl.ds(r0, NROW)], acc_hbm.at[c, pl.ds(r0, NROW)])


def _sc3_body(s_hbm, d_hbm, gs_hbm, z64_hbm, acc_hbm,
              acc_sh, sv0, dv0, sv1, dv1, gb0, gb1, sem0, sem1):
    c = lax.axis_index("c")
    t = lax.axis_index("s")
    wid = _wid()
    r0 = t * NROW
    pltpu.sync_copy(z64_hbm.at[pl.ds(r0, NROW)], acc_sh.at[pl.ds(r0, NROW)])
    plsc.subcore_barrier()
    base = wid * W1
    SLOT = ((sv0, dv0, gb0, sem0), (sv1, dv1, gb1, sem1))

    def fire(g, slot):
        sv, dv, gb, sem = SLOT[slot]
        off = base + g * K
        pltpu.sync_copy(s_hbm.at[pl.ds(off, K)], sv)
        pltpu.sync_copy(d_hbm.at[pl.ds(off, K)], dv)
        pltpu.async_copy(gs_hbm.at[sv], gb, sem)

    fire(0, 0)
    fire(1, 1)

    def chunk2(g2, _):
        for slot in range(2):
            g = g2 * 2 + slot
            sv, dv, gb, sem = SLOT[slot]
            pltpu.make_async_copy(gs_hbm.at[sv], gb, sem).wait()
            pltpu.sync_copy(gb, acc_sh.at[dv], add=True)

            @pl.when(g + 2 < NCH1)
            def _():
                fire(g + 2, slot)
        return 0

    lax.fori_loop(0, NCH1 // 2, chunk2, 0)
    plsc.subcore_barrier()
    pltpu.sync_copy(acc_sh.at[pl.ds(r0, NROW)], acc_hbm.at[c, pl.ds(r0, NROW)])


def _sc4_body(s_hbm, d_hbm, u_hbm, z_hbm,
              osrc_hbm, odst_hbm,
              sv0, dv0, sv1, dv1, uv, bs0, bd0, bs1, bd1,
              obs, obd, s0a, s0b, s1a, s1b):
    wid = _wid()
    base = wid * W4
    iot = lax.iota(_i32, 16)
    neg1 = jnp.full((16,), -1, _i32)
    zvec = jnp.zeros((16,), _f32)
    SLOT = ((sv0, dv0, bs0, bd0, s0a, s0b), (sv1, dv1, bs1, bd1, s1a, s1b))

    def fire(g, slot):
        sv, dv, bs, bd, sa, sb = SLOT[slot]
        off = base + g * K
        pltpu.sync_copy(s_hbm.at[pl.ds(off, K)], sv)
        pltpu.sync_copy(d_hbm.at[pl.ds(off, K)], dv)
        pltpu.async_copy(z_hbm.at[sv], bs, sa)
        pltpu.async_copy(z_hbm.at[dv], bd, sb)

    fire(0, 0)
    fire(1, 1)

    def chunk2(g2, _):
        for slot in range(2):
            g = g2 * 2 + slot
            off = base + g * K
            sv, dv, bs, bd, sa, sb = SLOT[slot]
            pltpu.sync_copy(u_hbm.at[pl.ds(off, K)], uv)
            pltpu.make_async_copy(z_hbm.at[sv], bs, sa).wait()
            pltpu.make_async_copy(z_hbm.at[dv], bd, sb).wait()

            for grp in range(K // 16):
                ridx = grp * 16 + iot

                def feat(cc, t):
                    col = jnp.full((16,), cc, _i32)
                    return t + (plsc.load_gather(bs, [ridx, col])
                                * plsc.load_gather(bd, [ridx, col]))

                t = lax.fori_loop(0, BOT, feat, zvec, unroll=8)
                sl = pl.ds(grp * 16, 16)
                p = 1.0 / (1.0 + jnp.exp(-t))
                keep = uv[sl] < p
                obs[sl] = jnp.where(keep, sv[sl], neg1)
                obd[sl] = jnp.where(keep, dv[sl], neg1)
            pltpu.sync_copy(obs, osrc_hbm.at[pl.ds(off, K)])
            pltpu.sync_copy(obd, odst_hbm.at[pl.ds(off, K)])

            @pl.when(g + 2 < NCH4)
            def _():
                fire(g + 2, slot)
        return 0

    lax.fori_loop(0, NCH4 // 2, chunk2, 0)


def _b(shape, imap):
    return pl.BlockSpec(shape, imap)


_ROW = lambda i: (i, 0)
_FIX = lambda i: (0, 0)
_FIX3 = lambda i: (0, i, 0)


def _tc0(x_pad, Wl, Wr):
    return pl.pallas_call(
        _tc0_body,
        grid=(NB,),
        in_specs=[_b((RB, 256), _ROW), _b((256, 256), _FIX),
                  _b((256, 256), _FIX)],
        out_specs=[_b((RB, 128), _ROW)] * 4,
        out_shape=[jax.ShapeDtypeStruct((NPAD, 128), _f32)] * 4,
    )(x_pad, Wl, Wr)


def _tc1a(acc_a, acc_b, x_pad, bias2d):
    return pl.pallas_call(
        _tc1a_body,
        grid=(NB,),
        in_specs=[_b((2, RB, 144), _FIX3), _b((2, RB, 144), _FIX3),
                  _b((RB, 256), _ROW), _b((1, 256), _FIX)],
        out_specs=[_b((RB, 256), _ROW), _b((RB, BOT), _ROW),
                   _b((1, 256), _FIX), _b((1, 256), _FIX)],
        out_shape=[jax.ShapeDtypeStruct((NPAD, 256), _f32),
                   jax.ShapeDtypeStruct((NPAD, BOT), _f32),
                   jax.ShapeDtypeStruct((1, 256), _f32),
                   jax.ShapeDtypeStruct((1, 256), _f32)],
    )(acc_a, acc_b, x_pad, bias2d)


def _tc1b(hmid, ssum, ssq, g1_2d, b1_2d, Wg, disb):
    return pl.pallas_call(
        _tc1b_body,
        grid=(NB,),
        in_specs=[_b((RB, 256), _ROW), _b((1, 256), _FIX),
                  _b((1, 256), _FIX), _b((1, 256), _FIX),
                  _b((1, 256), _FIX), _b((256, BOT), _FIX),
                  _b((RB, BOT), _ROW)],
        out_specs=[_b((RB, 256), _ROW), _b((RB, BOT), _ROW)],
        out_shape=[jax.ShapeDtypeStruct((NPAD, 256), _f32),
                   jax.ShapeDtypeStruct((NPAD, BOT), _f32)],
    )(hmid, ssum, ssq, g1_2d, b1_2d, Wg, disb)


def _tc2(acc2, disb, bg_2d, g2_2d, b2_2d):
    return pl.pallas_call(
        _tc2_body,
        out_shape=jax.ShapeDtypeStruct((NPAD, BOT), _f32),
    )(acc2, disb, bg_2d, g2_2d, b2_2d)


def _stage1(s_all, d_all, xla, xlb, xra, xrb, att_flat):
    return pl.kernel(
        _sc1_body,
        out_type=[jax.ShapeDtypeStruct((H, EP), _f32),
                  jax.ShapeDtypeStruct((512,), _f32)],
        mesh=plsc.VectorSubcoreMesh(**_MESH),
        compiler_params=pltpu.CompilerParams(needs_layout_passes=False, use_tc_tiling_on_sc=False),
        scratch_types=[
            pltpu.VMEM((K,), _i32), pltpu.VMEM((K,), _i32),
            pltpu.VMEM((K,), _i32), pltpu.VMEM((K,), _i32),
            pltpu.VMEM((K, 128), _f32), pltpu.VMEM((K, 128), _f32),
            pltpu.VMEM((K, 128), _f32), pltpu.VMEM((K, 128), _f32),
            pltpu.VMEM((K, 128), _f32), pltpu.VMEM((K, 128), _f32),
            pltpu.VMEM((K, 128), _f32), pltpu.VMEM((K, 128), _f32),
            pltpu.VMEM((HC,), _f32), pltpu.VMEM((H, K), _f32),
            pltpu.VMEM((16,), _f32),
            pltpu.SemaphoreType.DMA, pltpu.SemaphoreType.DMA,
            pltpu.SemaphoreType.DMA, pltpu.SemaphoreType.DMA,
            pltpu.SemaphoreType.DMA, pltpu.SemaphoreType.DMA,
            pltpu.SemaphoreType.DMA, pltpu.SemaphoreType.DMA,
        ],
    )(s_all, d_all, xla, xlb, xra, xrb, att_flat)


def _stage2(hoff, s_all, d_all, xh, logits, wmax, z144):
    return pl.kernel(
        functools.partial(_sc2_body, hoff),
        out_type=jax.ShapeDtypeStruct((2, NPAD, 144), _f32),
        mesh=plsc.VectorSubcoreMesh(**_MESH),
        compiler_params=pltpu.CompilerParams(needs_layout_passes=False, use_tc_tiling_on_sc=False),
        scratch_types=[
            pltpu.VMEM_SHARED((NPAD, 144), _f32),
            pltpu.VMEM((K,), _i32), pltpu.VMEM((K,), _i32),
            pltpu.VMEM((K,), _i32), pltpu.VMEM((K,), _i32),
            pltpu.VMEM((K, 128), _f32), pltpu.VMEM((K, 128), _f32),
            pltpu.VMEM((K,), _f32), pltpu.VMEM((K,), _f32),
            pltpu.VMEM((K,), _f32), pltpu.VMEM((K,), _f32),
            pltpu.VMEM((K, 144), _f32),
            pltpu.VMEM((512,), _f32),
            pltpu.SemaphoreType.DMA, pltpu.SemaphoreType.DMA,
        ],
    )(s_all, d_all, xh, logits, wmax, z144)


def _stage3(s_all, d_all, gs, z64):
    return pl.kernel(
        _sc3_body,
        out_type=jax.ShapeDtypeStruct((2, NPAD, BOT), _f32),
        mesh=plsc.VectorSubcoreMesh(**_MESH),
        compiler_params=pltpu.CompilerParams(needs_layout_passes=False, use_tc_tiling_on_sc=False),
        scratch_types=[
            pltpu.VMEM_SHARED((NPAD, BOT), _f32),
            pltpu.VMEM((K,), _i32), pltpu.VMEM((K,), _i32),
            pltpu.VMEM((K,), _i32), pltpu.VMEM((K,), _i32),
            pltpu.VMEM((K, BOT), _f32), pltpu.VMEM((K, BOT), _f32),
            pltpu.SemaphoreType.DMA, pltpu.SemaphoreType.DMA,
        ],
    )(s_all, d_all, gs, z64)


def _stage4(s2, d2, u2, z):
    return pl.kernel(
        _sc4_body,
        out_type=[jax.ShapeDtypeStruct((EP2,), _i32),
                  jax.ShapeDtypeStruct((EP2,), _i32)],
        mesh=plsc.VectorSubcoreMesh(**_MESH),
        compiler_params=pltpu.CompilerParams(needs_layout_passes=False, use_tc_tiling_on_sc=False),
        scratch_types=[
            pltpu.VMEM((K,), _i32), pltpu.VMEM((K,), _i32),
            pltpu.VMEM((K,), _i32), pltpu.VMEM((K,), _i32),
            pltpu.VMEM((K,), _f32),
            pltpu.VMEM((K, BOT), _f32), pltpu.VMEM((K, BOT), _f32),
            pltpu.VMEM((K, BOT), _f32), pltpu.VMEM((K, BOT), _f32),
            pltpu.VMEM((K,), _i32), pltpu.VMEM((K,), _i32),
            pltpu.SemaphoreType.DMA, pltpu.SemaphoreType.DMA,
            pltpu.SemaphoreType.DMA, pltpu.SemaphoreType.DMA,
        ],
    )(s2, d2, u2, z)


def kernel(x, edge_index, Wl, Wr, att, bias_gat, gamma1, beta1,
           Wg, bg, gamma2, beta2):
    src = edge_index[0]
    dst = edge_index[1]
    loops = jnp.arange(N, dtype=_i32)
    padi = jnp.full((EP - ETOT,), N, _i32)
    s_all = jnp.concatenate([src, loops, padi])
    d_all = jnp.concatenate([dst, loops, padi])
    padi2 = jnp.full((EP2 - E,), N, _i32)
    s2 = jnp.concatenate([src, padi2])
    d2 = jnp.concatenate([dst, padi2])
    u = jax.random.uniform(jax.random.key(42), (E,), _f32)
    u2 = jnp.concatenate([u, jnp.full((EP2 - E,), 2.0, _f32)])

    x_pad = jnp.pad(x, ((0, NPAD - N), (0, 0)))
    att_flat = att.reshape(HC)
    z144 = jnp.zeros((NPAD, 144), _f32)
    z64 = jnp.zeros((NPAD, BOT), _f32)

    xla, xlb, xra, xrb = _tc0(x_pad, Wl, Wr)

    logits, wmax = _stage1(s_all, d_all, xla, xlb, xra, xrb, att_flat)
    acc_a = _stage2(0, s_all, d_all, xla, logits, wmax, z144)
    acc_b = _stage2(2, s_all, d_all, xlb, logits, wmax, z144)

    hmid, disb, ssum, ssq = _tc1a(acc_a, acc_b, x_pad,
                                  bias_gat.reshape(1, 256))
    h_pad, gs = _tc1b(hmid, ssum, ssq, gamma1.reshape(1, 256),
                      beta1.reshape(1, 256), Wg, disb)

    acc2 = _stage3(s_all, d_all, gs, z64)

    z = _tc2(acc2, disb, bg.reshape(1, BOT), gamma2.reshape(1, BOT),
             beta2.reshape(1, BOT))

    osrc, odst = _stage4(s2, d2, u2, z)

    h = h_pad[:N]
    edge_index_out = jnp.stack([osrc[:E], odst[:E]])
    return (h, edge_index_out)


# speedup vs baseline: 1.1529x; 1.1529x over previous
import functools

import jax
import jax.numpy as jnp
from jax import lax
from jax.experimental import pallas as pl
from jax.experimental.pallas import tpu as pltpu
from jax.experimental.pallas import tpu_sc as plsc

N = 10000
E = 160000
D = 256
H = 4
C = 64
BOT = 64
HC = H * C

NPAD = 10112
NROW = NPAD // 16
ETOT = E + N
EP = 172032
EP2 = 163840
K = 64
W1 = EP // 32
NCH1 = W1 // K
W4 = EP2 // 32
NCH4 = W4 // K

_f32 = jnp.float32
_i32 = jnp.int32

_MESH = dict(core_axis_name="c", subcore_axis_name="s")


def _wid():
    return lax.axis_index("s") * 2 + lax.axis_index("c")


NB = 8
RB = NPAD // NB


def _tc0_body(x_ref, wl_ref, wr_ref, xla_ref, xlb_ref, xra_ref, xrb_ref):
    x = x_ref[...]
    xl = jnp.dot(x, wl_ref[...], preferred_element_type=_f32,
                 precision=lax.Precision.DEFAULT)
    xr = jnp.dot(x, wr_ref[...], preferred_element_type=_f32,
                 precision=lax.Precision.DEFAULT)
    xla_ref[...] = xl[:, :128]
    xlb_ref[...] = xl[:, 128:]
    xra_ref[...] = xr[:, :128]
    xrb_ref[...] = xr[:, 128:]


def _tc1a_body(aa_ref, ab_ref, x_ref, bias_ref,
               hmid_ref, disb_ref, ssum_ref, ssq_ref):
    i = pl.program_id(0)

    @pl.when(i == 0)
    def _():
        ssum_ref[...] = jnp.zeros_like(ssum_ref)
        ssq_ref[...] = jnp.zeros_like(ssq_ref)

    a = aa_ref[0] + aa_ref[1]
    b = ab_ref[0] + ab_ref[1]
    d0 = a[:, 128:129]
    d1 = a[:, 129:130]
    d2 = b[:, 128:129]
    d3 = b[:, 129:130]
    deg = a[:, 130:131]
    gat = jnp.concatenate([
        jnp.where(d0 > 0, a[:, :64] / d0, 0.0),
        jnp.where(d1 > 0, a[:, 64:128] / d1, 0.0),
        jnp.where(d2 > 0, b[:, :64] / d2, 0.0),
        jnp.where(d3 > 0, b[:, 64:128] / d3, 0.0),
    ], axis=1) + bias_ref[...]
    hmid = jnp.maximum(gat, 0.0) + x_ref[...]
    hmid_ref[...] = hmid
    dis = jnp.where(deg > 0, lax.rsqrt(deg), 0.0)
    disb_ref[...] = jnp.broadcast_to(dis, (RB, BOT))
    rows = i * RB + lax.broadcasted_iota(_i32, (RB, 1), 0)
    hm = jnp.where(rows < N, hmid, 0.0)
    ssum_ref[...] += jnp.sum(hm, axis=0, keepdims=True)
    ssq_ref[...] += jnp.sum(hm * hm, axis=0, keepdims=True)


def _tc1b_body(hmid_ref, ssum_ref, ssq_ref, g1_ref, b1_ref, wg_ref, disb_ref,
               h_ref, gs_ref):
    mean = ssum_ref[...] / N
    var = ssq_ref[...] / N - mean * mean
    h = (hmid_ref[...] - mean) * lax.rsqrt(var + 1e-5) * g1_ref[...] \
        + b1_ref[...]
    h_ref[...] = h
    g1 = jnp.dot(h, wg_ref[...], preferred_element_type=_f32,
                 precision=lax.Precision.DEFAULT)
    gs_ref[...] = g1 * disb_ref[...]


def _tc2_body(a2_ref, disb_ref, bg_ref, g2_ref, b2_ref, z_ref):
    og = (a2_ref[0] + a2_ref[1]) * disb_ref[...] + bg_ref[...]
    r = jnp.maximum(og, 0.0)
    rows = lax.broadcasted_iota(_i32, (NPAD, 1), 0)
    mask = rows < N
    rm = jnp.where(mask, r, 0.0)
    mean = jnp.sum(rm, axis=0, keepdims=True) / N
    dev = jnp.where(mask, r - mean, 0.0)
    var = jnp.sum(dev * dev, axis=0, keepdims=True) / N
    z_ref[...] = (r - mean) * lax.rsqrt(var + 1e-5) * g2_ref[...] \
        + b2_ref[...]


def _sc1_body(s_hbm, d_hbm, xla_hbm, xlb_hbm, xra_hbm, xrb_hbm, att_hbm,
              logit_hbm, wmax_hbm,
              sv0, dv0, sv1, dv1,
              bla0, blb0, bra0, brb0, bla1, blb1, bra1, brb1,
              attv, lb, mbuf,
              s0a, s0b, s0c, s0d, s1a, s1b, s1c, s1d):
    wid = _wid()
    base = wid * W1
    pltpu.sync_copy(att_hbm, attv)
    iot = lax.iota(_i32, 16)
    zvec = jnp.zeros((16,), _f32)
    SLOT = (
        (sv0, dv0, (bla0, blb0, bra0, brb0), (s0a, s0b, s0c, s0d)),
        (sv1, dv1, (bla1, blb1, bra1, brb1), (s1a, s1b, s1c, s1d)),
    )

    def fire(g, slot):
        sv, dv, bufs, sems = SLOT[slot]
        off = base + g * K
        pltpu.sync_copy(s_hbm.at[pl.ds(off, K)], sv)
        pltpu.sync_copy(d_hbm.at[pl.ds(off, K)], dv)
        pltpu.async_copy(xla_hbm.at[sv], bufs[0], sems[0])
        pltpu.async_copy(xlb_hbm.at[sv], bufs[1], sems[1])
        pltpu.async_copy(xra_hbm.at[dv], bufs[2], sems[2])
        pltpu.async_copy(xrb_hbm.at[dv], bufs[3], sems[3])

    def drain(slot):
        sv, dv, bufs, sems = SLOT[slot]
        pltpu.make_async_copy(xla_hbm.at[sv], bufs[0], sems[0]).wait()
        pltpu.make_async_copy(xlb_hbm.at[sv], bufs[1], sems[1]).wait()
        pltpu.make_async_copy(xra_hbm.at[dv], bufs[2], sems[2]).wait()
        pltpu.make_async_copy(xrb_hbm.at[dv], bufs[3], sems[3]).wait()

    fire(0, 0)
    fire(1, 1)

    def chunk2(g2, mvec):
        for slot in range(2):
            g = g2 * 2 + slot
            off = base + g * K
            drain(slot)
            bla, blb, bra, brb = SLOT[slot][2]
            for h in range(H):
                bl, br = (bla, bra) if h < 2 else (blb, brb)
                cbase = (h % 2) * 64

                def feat(c2, accs):
                    asp = plsc.load_gather(
                        attv, [jnp.full((16,), h * 64 + c2, _i32)])
                    col = jnp.full((16,), cbase + c2, _i32)
                    out = []
                    for grp in range(K // 16):
                        ridx = grp * 16 + iot
                        u = (plsc.load_gather(bl, [ridx, col])
                             + plsc.load_gather(br, [ridx, col]))
                        u = jnp.maximum(u, 0.2 * u)
                        out.append(accs[grp] + asp * u)
                    return tuple(out)

                accs = lax.fori_loop(0, C, feat, (zvec,) * (K // 16),
                                     unroll=8)
                for grp in range(K // 16):
                    lb[h, pl.ds(grp * 16, 16)] = accs[grp]
                    mvec = jnp.maximum(mvec, accs[grp])
            pltpu.sync_copy(lb, logit_hbm.at[:, pl.ds(off, K)])

            @pl.when(g + 2 < NCH1)
            def _():
                fire(g + 2, slot)
        return mvec

    mvec = lax.fori_loop(0, NCH1 // 2, chunk2,
                         jnp.full((16,), -3.0e38, _f32))
    mbuf[...] = mvec
    pltpu.sync_copy(mbuf, wmax_hbm.at[pl.ds(wid * 16, 16)])


def _sc2_body(hoff, s_hbm, d_hbm, xh_hbm, logit_hbm, wmax_hbm, z144_hbm,
              acc_hbm,
              acc_sh, sv0, dv0, sv1, dv1, bx0, bx1,
              lb0, lb1, wb0, wb1, msg, mxv, sem0, sem1):
    c = lax.axis_index("c")
    t = lax.axis_index("s")
    wid = _wid()
    pltpu.sync_copy(wmax_hbm, mxv)
    mv = mxv[pl.ds(0, 16)]
    for v in range(1, 32):
        mv = jnp.maximum(mv, mxv[pl.ds(v * 16, 16)])
    gmax = jnp.max(mv)

    r0 = t * NROW
    pltpu.sync_copy(z144_hbm.at[pl.ds(r0, NROW)], acc_sh.at[pl.ds(r0, NROW)])
    plsc.subcore_barrier()

    base = wid * W1
    iot = lax.iota(_i32, 16)
    SLOT = ((sv0, dv0, bx0, sem0), (sv1, dv1, bx1, sem1))

    def fire(g, slot):
        sv, dv, bx, sem = SLOT[slot]
        off = base + g * K
        pltpu.sync_copy(s_hbm.at[pl.ds(off, K)], sv)
        pltpu.sync_copy(d_hbm.at[pl.ds(off, K)], dv)
        pltpu.async_copy(xh_hbm.at[sv], bx, sem)

    fire(0, 0)
    fire(1, 1)

    def chunk2(g2, _):
        for slot in range(2):
            g = g2 * 2 + slot
            off = base + g * K
            sv, dv, bx, sem = SLOT[slot]
            pltpu.sync_copy(logit_hbm.at[hoff, pl.ds(off, K)], lb0)
            pltpu.sync_copy(logit_hbm.at[hoff + 1, pl.ds(off, K)], lb1)
            pltpu.make_async_copy(xh_hbm.at[sv], bx, sem).wait()
            for v in range(K // 16):
                sl = pl.ds(v * 16, 16)
                wb0[sl] = jnp.exp(lb0[sl] - gmax)
                wb1[sl] = jnp.exp(lb1[sl] - gmax)

            def edge(j, _):
                jf = jnp.full((16,), j, _i32)
                w0 = plsc.load_gather(wb0, [jf])
                w1 = plsc.load_gather(wb1, [jf])
                for v in range(4):
                    msg[j, pl.ds(v * 16, 16)] = bx[j, pl.ds(v * 16, 16)] * w0
                for v in range(4, 8):
                    msg[j, pl.ds(v * 16, 16)] = bx[j, pl.ds(v * 16, 16)] * w1
                tail = jnp.where(iot == 0, w0,
                                 jnp.where(iot == 1, w1,
                                           jnp.where(iot == 2, 1.0, 0.0)))
                msg[j, pl.ds(128, 16)] = tail
                return 0

            lax.fori_loop(0, K, edge, 0, unroll=4)
            pltpu.sync_copy(msg, acc_sh.at[dv], add=True)

            @pl.when(g + 2 < NCH1)
            def _():
                fire(g + 2, slot)
        return 0

    lax.fori_loop(0, NCH1 // 2, chunk2, 0)
    plsc.subcore_barrier()
    pltpu.sync_copy(acc_sh.at[pl.ds(r0, NROW)], acc_hbm.at[c, pl.ds(r0, NROW)])


def _sc3_body(s_hbm, d_hbm, gs_hbm, z64_hbm, acc_hbm,
              acc_sh, sv0, dv0, sv1, dv1, gb0, gb1, sem0, sem1):
    c = lax.axis_index("c")
    t = lax.axis_index("s")
    wid = _wid()
    r0 = t * NROW
    pltpu.sync_copy(z64_hbm.at[pl.ds(r0, NROW)], acc_sh.at[pl.ds(r0, NROW)])
    plsc.subcore_barrier()
    base = wid * W1
    SLOT = ((sv0, dv0, gb0, sem0), (sv1, dv1, gb1, sem1))

    def fire(g, slot):
        sv, dv, gb, sem = SLOT[slot]
        off = base + g * K
        pltpu.sync_copy(s_hbm.at[pl.ds(off, K)], sv)
        pltpu.sync_copy(d_hbm.at[pl.ds(off, K)], dv)
        pltpu.async_copy(gs_hbm.at[sv], gb, sem)

    fire(0, 0)
    fire(1, 1)

    def chunk2(g2, _):
        for slot in range(2):
            g = g2 * 2 + slot
            sv, dv, gb, sem = SLOT[slot]
            pltpu.make_async_copy(gs_hbm.at[sv], gb, sem).wait()
            pltpu.sync_copy(gb, acc_sh.at[dv], add=True)

            @pl.when(g + 2 < NCH1)
            def _():
                fire(g + 2, slot)
        return 0

    lax.fori_loop(0, NCH1 // 2, chunk2, 0)
    plsc.subcore_barrier()
    pltpu.sync_copy(acc_sh.at[pl.ds(r0, NROW)], acc_hbm.at[c, pl.ds(r0, NROW)])


def _sc4_body(s_hbm, d_hbm, u_hbm, z_hbm,
              osrc_hbm, odst_hbm,
              sv0, dv0, sv1, dv1, uv, bs0, bd0, bs1, bd1,
              obs, obd, s0a, s0b, s1a, s1b):
    wid = _wid()
    base = wid * W4
    iot = lax.iota(_i32, 16)
    neg1 = jnp.full((16,), -1, _i32)
    zvec = jnp.zeros((16,), _f32)
    SLOT = ((sv0, dv0, bs0, bd0, s0a, s0b), (sv1, dv1, bs1, bd1, s1a, s1b))

    def fire(g, slot):
        sv, dv, bs, bd, sa, sb = SLOT[slot]
        off = base + g * K
        pltpu.sync_copy(s_hbm.at[pl.ds(off, K)], sv)
        pltpu.sync_copy(d_hbm.at[pl.ds(off, K)], dv)
        pltpu.async_copy(z_hbm.at[sv], bs, sa)
        pltpu.async_copy(z_hbm.at[dv], bd, sb)

    fire(0, 0)
    fire(1, 1)

    def chunk2(g2, _):
        for slot in range(2):
            g = g2 * 2 + slot
            off = base + g * K
            sv, dv, bs, bd, sa, sb = SLOT[slot]
            pltpu.sync_copy(u_hbm.at[pl.ds(off, K)], uv)
            pltpu.make_async_copy(z_hbm.at[sv], bs, sa).wait()
            pltpu.make_async_copy(z_hbm.at[dv], bd, sb).wait()

            for grp in range(K // 16):
                ridx = grp * 16 + iot

                def feat(cc, t):
                    col = jnp.full((16,), cc, _i32)
                    return t + (plsc.load_gather(bs, [ridx, col])
                                * plsc.load_gather(bd, [ridx, col]))

                t = lax.fori_loop(0, BOT, feat, zvec, unroll=8)
                sl = pl.ds(grp * 16, 16)
                p = 1.0 / (1.0 + jnp.exp(-t))
                keep = uv[sl] < p
                obs[sl] = jnp.where(keep, sv[sl], neg1)
                obd[sl] = jnp.where(keep, dv[sl], neg1)
            pltpu.sync_copy(obs, osrc_hbm.at[pl.ds(off, K)])
            pltpu.sync_copy(obd, odst_hbm.at[pl.ds(off, K)])

            @pl.when(g + 2 < NCH4)
            def _():
                fire(g + 2, slot)
        return 0

    lax.fori_loop(0, NCH4 // 2, chunk2, 0)


def _b(shape, imap):
    return pl.BlockSpec(shape, imap)


_ROW = lambda i: (i, 0)
_FIX = lambda i: (0, 0)
_FIX3 = lambda i: (0, i, 0)


def _tc0(x_pad, Wl, Wr):
    return pl.pallas_call(
        _tc0_body,
        grid=(NB,),
        in_specs=[_b((RB, 256), _ROW), _b((256, 256), _FIX),
                  _b((256, 256), _FIX)],
        out_specs=[_b((RB, 128), _ROW)] * 4,
        out_shape=[jax.ShapeDtypeStruct((NPAD, 128), _f32)] * 4,
    )(x_pad, Wl, Wr)


def _tc1a(acc_a, acc_b, x_pad, bias2d):
    return pl.pallas_call(
        _tc1a_body,
        grid=(NB,),
        in_specs=[_b((2, RB, 144), _FIX3), _b((2, RB, 144), _FIX3),
                  _b((RB, 256), _ROW), _b((1, 256), _FIX)],
        out_specs=[_b((RB, 256), _ROW), _b((RB, BOT), _ROW),
                   _b((1, 256), _FIX), _b((1, 256), _FIX)],
        out_shape=[jax.ShapeDtypeStruct((NPAD, 256), _f32),
                   jax.ShapeDtypeStruct((NPAD, BOT), _f32),
                   jax.ShapeDtypeStruct((1, 256), _f32),
                   jax.ShapeDtypeStruct((1, 256), _f32)],
    )(acc_a, acc_b, x_pad, bias2d)


def _tc1b(hmid, ssum, ssq, g1_2d, b1_2d, Wg, disb):
    return pl.pallas_call(
        _tc1b_body,
        grid=(NB,),
        in_specs=[_b((RB, 256), _ROW), _b((1, 256), _FIX),
                  _b((1, 256), _FIX), _b((1, 256), _FIX),
                  _b((1, 256), _FIX), _b((256, BOT), _FIX),
                  _b((RB, BOT), _ROW)],
        out_specs=[_b((RB, 256), _ROW), _b((RB, BOT), _ROW)],
        out_shape=[jax.ShapeDtypeStruct((NPAD, 256), _f32),
                   jax.ShapeDtypeStruct((NPAD, BOT), _f32)],
    )(hmid, ssum, ssq, g1_2d, b1_2d, Wg, disb)


def _tc2(acc2, disb, bg_2d, g2_2d, b2_2d):
    return pl.pallas_call(
        _tc2_body,
        out_shape=jax.ShapeDtypeStruct((NPAD, BOT), _f32),
    )(acc2, disb, bg_2d, g2_2d, b2_2d)


def _stage1(s_all, d_all, xla, xlb, xra, xrb, att_flat):
    return pl.kernel(
        _sc1_body,
        out_type=[jax.ShapeDtypeStruct((H, EP), _f32),
                  jax.ShapeDtypeStruct((512,), _f32)],
        mesh=plsc.VectorSubcoreMesh(**_MESH),
        compiler_params=pltpu.CompilerParams(needs_layout_passes=False, use_tc_tiling_on_sc=False),
        scratch_types=[
            pltpu.VMEM((K,), _i32), pltpu.VMEM((K,), _i32),
            pltpu.VMEM((K,), _i32), pltpu.VMEM((K,), _i32),
            pltpu.VMEM((K, 128), _f32), pltpu.VMEM((K, 128), _f32),
            pltpu.VMEM((K, 128), _f32), pltpu.VMEM((K, 128), _f32),
            pltpu.VMEM((K, 128), _f32), pltpu.VMEM((K, 128), _f32),
            pltpu.VMEM((K, 128), _f32), pltpu.VMEM((K, 128), _f32),
            pltpu.VMEM((HC,), _f32), pltpu.VMEM((H, K), _f32),
            pltpu.VMEM((16,), _f32),
            pltpu.SemaphoreType.DMA, pltpu.SemaphoreType.DMA,
            pltpu.SemaphoreType.DMA, pltpu.SemaphoreType.DMA,
            pltpu.SemaphoreType.DMA, pltpu.SemaphoreType.DMA,
            pltpu.SemaphoreType.DMA, pltpu.SemaphoreType.DMA,
        ],
    )(s_all, d_all, xla, xlb, xra, xrb, att_flat)


def _stage2(hoff, s_all, d_all, xh, logits, wmax, z144):
    return pl.kernel(
        functools.partial(_sc2_body, hoff),
        out_type=jax.ShapeDtypeStruct((2, NPAD, 144), _f32),
        mesh=plsc.VectorSubcoreMesh(**_MESH),
        compiler_params=pltpu.CompilerParams(needs_layout_passes=False, use_tc_tiling_on_sc=False),
        scratch_types=[
            pltpu.VMEM_SHARED((NPAD, 144), _f32),
            pltpu.VMEM((K,), _i32), pltpu.VMEM((K,), _i32),
            pltpu.VMEM((K,), _i32), pltpu.VMEM((K,), _i32),
            pltpu.VMEM((K, 128), _f32), pltpu.VMEM((K, 128), _f32),
            pltpu.VMEM((K,), _f32), pltpu.VMEM((K,), _f32),
            pltpu.VMEM((K,), _f32), pltpu.VMEM((K,), _f32),
            pltpu.VMEM((K, 144), _f32),
            pltpu.VMEM((512,), _f32),
            pltpu.SemaphoreType.DMA, pltpu.SemaphoreType.DMA,
        ],
    )(s_all, d_all, xh, logits, wmax, z144)


def _stage3(s_all, d_all, gs, z64):
    return pl.kernel(
        _sc3_body,
        out_type=jax.ShapeDtypeStruct((2, NPAD, BOT), _f32),
        mesh=plsc.VectorSubcoreMesh(**_MESH),
        compiler_params=pltpu.CompilerParams(needs_layout_passes=False, use_tc_tiling_on_sc=False),
        scratch_types=[
            pltpu.VMEM_SHARED((NPAD, BOT), _f32),
            pltpu.VMEM((K,), _i32), pltpu.VMEM((K,), _i32),
            pltpu.VMEM((K,), _i32), pltpu.VMEM((K,), _i32),
            pltpu.VMEM((K, BOT), _f32), pltpu.VMEM((K, BOT), _f32),
            pltpu.SemaphoreType.DMA, pltpu.SemaphoreType.DMA,
        ],
    )(s_all, d_all, gs, z64)


def _stage4(s2, d2, u2, z):
    return pl.kernel(
        _sc4_body,
        out_type=[jax.ShapeDtypeStruct((EP2,), _i32),
                  jax.ShapeDtypeStruct((EP2,), _i32)],
        mesh=plsc.VectorSubcoreMesh(**_MESH),
        compiler_params=pltpu.CompilerParams(needs_layout_passes=False, use_tc_tiling_on_sc=False),
        scratch_types=[
            pltpu.VMEM((K,), _i32), pltpu.VMEM((K,), _i32),
            pltpu.VMEM((K,), _i32), pltpu.VMEM((K,), _i32),
            pltpu.VMEM((K,), _f32),
            pltpu.VMEM((K, BOT), _f32), pltpu.VMEM((K, BOT), _f32),
            pltpu.VMEM((K, BOT), _f32), pltpu.VMEM((K, BOT), _f32),
            pltpu.VMEM((K,), _i32), pltpu.VMEM((K,), _i32),
            pltpu.SemaphoreType.DMA, pltpu.SemaphoreType.DMA,
            pltpu.SemaphoreType.DMA, pltpu.SemaphoreType.DMA,
        ],
    )(s2, d2, u2, z)


def kernel(x, edge_index, Wl, Wr, att, bias_gat, gamma1, beta1,
           Wg, bg, gamma2, beta2):
    src = edge_index[0]
    dst = edge_index[1]
    loops = jnp.arange(N, dtype=_i32)
    padi = jnp.full((EP - ETOT,), N, _i32)
    s_all = jnp.concatenate([src, loops, padi])
    d_all = jnp.concatenate([dst, loops, padi])
    padi2 = jnp.full((EP2 - E,), N, _i32)
    s2 = jnp.concatenate([src, padi2])
    d2 = jnp.concatenate([dst, padi2])
    u = jax.random.uniform(jax.random.key(42), (E,), _f32)
    u2 = jnp.concatenate([u, jnp.full((EP2 - E,), 2.0, _f32)])

    x_pad = jnp.pad(x, ((0, NPAD - N), (0, 0)))
    att_flat = att.reshape(HC)
    z144 = jnp.zeros((NPAD, 144), _f32)
    z64 = jnp.zeros((NPAD, BOT), _f32)

    xla, xlb, xra, xrb = _tc0(x_pad, Wl, Wr)

    logits, wmax = _stage1(s_all, d_all, xla, xlb, xra, xrb, att_flat)
    acc_a = _stage2(0, s_all, d_all, xla, logits, wmax, z144)
    acc_b = _stage2(2, s_all, d_all, xlb, logits, wmax, z144)

    hmid, disb, ssum, ssq = _tc1a(acc_a, acc_b, x_pad,
                                  bias_gat.reshape(1, 256))
    h_pad, gs = _tc1b(hmid, ssum, ssq, gamma1.reshape(1, 256),
                      beta1.reshape(1, 256), Wg, disb)

    acc2 = _stage3(s_all, d_all, gs, z64)

    z = _tc2(acc2, disb, bg.reshape(1, BOT), gamma2.reshape(1, BOT),
             beta2.reshape(1, BOT))

    osrc, odst = _stage4(s2, d2, u2, z)

    h = h_pad[:N]
    edge_index_out = jnp.stack([osrc[:E], odst[:E]])
    return (h, edge_index_out)
